# Initial kernel scaffold; baseline (speedup 1.0000x reference)
import jax
import jax.numpy as jnp
from jax.experimental import pallas as pl


def kernel(idx, weight):
    raise NotImplementedError("write your pallas kernel here")


# speedup vs baseline: 1.1046x; 1.1046x over previous
import functools

import jax
import jax.numpy as jnp
from jax import lax
from jax.experimental import pallas as pl
from jax.experimental.pallas import tpu as pltpu
from jax.experimental.pallas import tpu_sc as plsc

NC = 2
NS = 16
NW = NC * NS

S = 128
K = 10
GROUP = S * K


@functools.partial(jax.jit, static_argnames=("B", "D"))
def _sc_gather(idx2d, weight, B, D):
    nrows_w = B // NW
    nsub_w = nrows_w // S
    G = nsub_w // K

    mesh = plsc.VectorSubcoreMesh(core_axis_name="c", subcore_axis_name="s")

    @functools.partial(
        pl.kernel,
        mesh=mesh,
        out_type=jax.ShapeDtypeStruct((B, D), jnp.float32),
        scratch_types=[
            pltpu.VMEM((nsub_w, S), jnp.int32),
            pltpu.VMEM((GROUP, D), jnp.float32),
            pltpu.SemaphoreType.DMA,
            pltpu.SemaphoreType.DMA,
        ],
        compiler_params=pltpu.CompilerParams(use_tc_tiling_on_sc=False),
    )
    def k(idx_hbm, w_hbm, out_hbm, idx_v, buf, sem_g, sem_w):
        wid = lax.axis_index("s") * NC + lax.axis_index("c")
        pltpu.sync_copy(idx_hbm.at[pl.ds(wid * nsub_w, nsub_w)], idx_v)
        row0 = wid * nrows_w

        def body(g, carry):
            copies = [
                pltpu.async_copy(
                    w_hbm.at[idx_v.at[g * K + kk]],
                    buf.at[pl.ds(kk * S, S)],
                    sem_g,
                )
                for kk in range(K)
            ]
            for c in copies:
                c.wait()
            pltpu.async_copy(
                buf, out_hbm.at[pl.ds(row0 + g * GROUP, GROUP)], sem_w
            ).wait()
            return carry

        lax.fori_loop(0, G, body, 0)

    return k(idx2d, weight)


def kernel(idx, weight):
    B = idx.shape[0] * idx.shape[1]
    D = weight.shape[1]
    idx2d = idx.astype(jnp.int32).reshape(B // S, S)
    out = _sc_gather(idx2d, weight, B, D)
    return out.reshape(idx.shape + (D,))


# speedup vs baseline: 1.1090x; 1.0039x over previous
import functools

import jax
import jax.numpy as jnp
from jax import lax
from jax.experimental import pallas as pl
from jax.experimental.pallas import tpu as pltpu
from jax.experimental.pallas import tpu_sc as plsc

NC = 2
NS = 16
NW = NC * NS

S = 128
K = 10
GROUP = S * K


@functools.partial(jax.jit, static_argnames=("B", "D"))
def _sc_gather(idx2d, weight, B, D):
    nrows_w = B // NW
    nsub_w = nrows_w // S
    G = nsub_w // K

    mesh = plsc.VectorSubcoreMesh(core_axis_name="c", subcore_axis_name="s")

    @functools.partial(
        pl.kernel,
        mesh=mesh,
        out_type=jax.ShapeDtypeStruct((B, D), jnp.float32),
        scratch_types=[
            pltpu.VMEM((nsub_w, S), jnp.int32),
            pltpu.VMEM((GROUP, D), jnp.float32),
            pltpu.VMEM((GROUP, D), jnp.float32),
            pltpu.SemaphoreType.DMA,
            pltpu.SemaphoreType.DMA,
            pltpu.SemaphoreType.DMA,
            pltpu.SemaphoreType.DMA,
        ],
        compiler_params=pltpu.CompilerParams(use_tc_tiling_on_sc=False),
    )
    def k(idx_hbm, w_hbm, out_hbm, idx_v, buf0, buf1, sg0, sg1, sw0, sw1):
        wid = lax.axis_index("s") * NC + lax.axis_index("c")
        pltpu.sync_copy(idx_hbm.at[pl.ds(wid * nsub_w, nsub_w)], idx_v)
        row0 = wid * nrows_w

        def fire_gathers(g, buf, sem):
            for kk in range(K):
                pltpu.async_copy(
                    w_hbm.at[idx_v.at[g * K + kk]],
                    buf.at[pl.ds(kk * S, S)],
                    sem,
                )

        def drain_gathers(buf, sem):
            pltpu.make_async_copy(out_hbm.at[pl.ds(0, GROUP)], buf, sem).wait()

        def fire_write(g, buf, sem):
            pltpu.async_copy(
                buf, out_hbm.at[pl.ds(row0 + g * GROUP, GROUP)], sem
            )

        def drain_write(buf, sem):
            pltpu.make_async_copy(out_hbm.at[pl.ds(0, GROUP)], buf, sem).wait()

        fire_gathers(0, buf0, sg0)

        drain_gathers(buf0, sg0)
        fire_write(0, buf0, sw0)
        fire_gathers(1, buf1, sg1)
        drain_gathers(buf1, sg1)
        fire_write(1, buf1, sw1)
        drain_write(buf0, sw0)
        fire_gathers(2, buf0, sg0)

        def body(i, carry):
            g = 2 * i
            drain_gathers(buf0, sg0)
            fire_write(g, buf0, sw0)
            drain_write(buf1, sw1)
            fire_gathers(g + 1, buf1, sg1)
            drain_gathers(buf1, sg1)
            fire_write(g + 1, buf1, sw1)
            drain_write(buf0, sw0)
            fire_gathers(g + 2, buf0, sg0)
            return carry

        lax.fori_loop(1, G // 2 - 1, body, 0)

        drain_gathers(buf0, sg0)
        fire_write(G - 2, buf0, sw0)
        drain_write(buf1, sw1)
        fire_gathers(G - 1, buf1, sg1)
        drain_gathers(buf1, sg1)
        fire_write(G - 1, buf1, sw1)
        drain_write(buf0, sw0)
        drain_write(buf1, sw1)

    return k(idx2d, weight)


def kernel(idx, weight):
    B = idx.shape[0] * idx.shape[1]
    D = weight.shape[1]
    idx2d = idx.astype(jnp.int32).reshape(B // S, S)
    out = _sc_gather(idx2d, weight, B, D)
    return out.reshape(idx.shape + (D,))


# speedup vs baseline: 1.1115x; 1.0023x over previous
import functools

import jax
import jax.numpy as jnp
from jax import lax
from jax.experimental import pallas as pl
from jax.experimental.pallas import tpu as pltpu
from jax.experimental.pallas import tpu_sc as plsc

NC = 2
NS = 16
NW = NC * NS

S = 128
K = 10
GROUP = S * K


@functools.partial(jax.jit, static_argnames=("B", "D"))
def _sc_gather(idx2d, weight, B, D):
    nrows_w = B // NW
    nsub_w = nrows_w // S
    G = nsub_w // K

    mesh = plsc.VectorSubcoreMesh(core_axis_name="c", subcore_axis_name="s")

    @functools.partial(
        pl.kernel,
        mesh=mesh,
        out_type=jax.ShapeDtypeStruct((B, D), jnp.float32),
        scratch_types=[
            pltpu.VMEM((nsub_w, S), jnp.int32),
            pltpu.VMEM((GROUP, D), jnp.float32),
            pltpu.VMEM((GROUP, D), jnp.float32),
            pltpu.SemaphoreType.DMA,
            pltpu.SemaphoreType.DMA,
            pltpu.SemaphoreType.DMA,
            pltpu.SemaphoreType.DMA,
        ],
        compiler_params=pltpu.CompilerParams(use_tc_tiling_on_sc=False),
    )
    def k(idx_hbm, w_hbm, out_hbm, idx_v, buf0, buf1, sg0, sg1, sw0, sw1):
        wid = lax.axis_index("s") * NC + lax.axis_index("c")
        pltpu.sync_copy(idx_hbm.at[pl.ds(wid * nsub_w, nsub_w)], idx_v)
        row0 = wid * nrows_w

        def fire_gathers(g, buf, sem):
            for kk in range(K):
                pltpu.async_copy(
                    w_hbm.at[idx_v.at[g * K + kk]],
                    buf.at[pl.ds(kk * S, S)],
                    sem,
                )

        def drain_gathers(buf, sem):
            pltpu.make_async_copy(out_hbm.at[pl.ds(0, GROUP)], buf, sem).wait()

        def fire_write(g, buf, sem):
            pltpu.async_copy(
                buf, out_hbm.at[pl.ds(row0 + g * GROUP, GROUP)], sem
            )

        def drain_write(buf, sem):
            pltpu.make_async_copy(out_hbm.at[pl.ds(0, GROUP)], buf, sem).wait()

        fire_gathers(0, buf0, sg0)

        fire_gathers(1, buf1, sg1)
        drain_gathers(buf0, sg0)
        fire_write(0, buf0, sw0)
        drain_write(buf0, sw0)
        fire_gathers(2, buf0, sg0)
        drain_gathers(buf1, sg1)
        fire_write(1, buf1, sw1)

        def body(i, carry):
            g = 2 * i
            drain_write(buf1, sw1)
            fire_gathers(g + 1, buf1, sg1)
            drain_gathers(buf0, sg0)
            fire_write(g, buf0, sw0)
            drain_write(buf0, sw0)
            fire_gathers(g + 2, buf0, sg0)
            drain_gathers(buf1, sg1)
            fire_write(g + 1, buf1, sw1)
            return carry

        lax.fori_loop(1, G // 2 - 1, body, 0)

        drain_write(buf1, sw1)
        fire_gathers(G - 1, buf1, sg1)
        drain_gathers(buf0, sg0)
        fire_write(G - 2, buf0, sw0)
        drain_gathers(buf1, sg1)
        fire_write(G - 1, buf1, sw1)
        drain_write(buf0, sw0)
        drain_write(buf1, sw1)

    return k(idx2d, weight)


def kernel(idx, weight):
    B = idx.shape[0] * idx.shape[1]
    D = weight.shape[1]
    idx2d = idx.astype(jnp.int32).reshape(B // S, S)
    out = _sc_gather(idx2d, weight, B, D)
    return out.reshape(idx.shape + (D,))


# speedup vs baseline: 1.8025x; 1.6217x over previous
import functools

import jax
import jax.numpy as jnp
from jax import lax
from jax.experimental import pallas as pl
from jax.experimental.pallas import tpu as pltpu
from jax.experimental.pallas import tpu_sc as plsc

NC = 2
NS = 16
NW = NC * NS

R = 16


@functools.partial(jax.jit, static_argnames=("N", "Q", "D"))
def _sc_gather(idx, weight, N, Q, D):
    nrows_w = N // NW
    G = nrows_w // R
    gbytes = R * Q * D * 4

    mesh = plsc.VectorSubcoreMesh(core_axis_name="c", subcore_axis_name="s")

    @functools.partial(
        pl.kernel,
        mesh=mesh,
        out_type=jax.ShapeDtypeStruct((N, Q, D), jnp.float32),
        scratch_types=[
            pltpu.VMEM((nrows_w, Q), jnp.int32),
            pltpu.VMEM((R, Q, D), jnp.float32),
            pltpu.VMEM((R, Q, D), jnp.float32),
            pltpu.SemaphoreType.DMA,
            pltpu.SemaphoreType.DMA,
            pltpu.SemaphoreType.DMA,
            pltpu.SemaphoreType.DMA,
        ],
        compiler_params=pltpu.CompilerParams(use_tc_tiling_on_sc=False),
    )
    def k(idx_hbm, w_hbm, out_hbm, idx_v, buf0, buf1, sg0, sg1, sw0, sw1):
        wid = lax.axis_index("s") * NC + lax.axis_index("c")
        row0 = wid * nrows_w
        pltpu.sync_copy(idx_hbm.at[pl.ds(row0, nrows_w)], idx_v)

        def fire_gathers(g, buf, sem):
            def fire_one(r, carry):
                pltpu.async_copy(
                    w_hbm.at[idx_v.at[g * R + r]], buf.at[r], sem
                )
                return carry

            lax.fori_loop(0, R, fire_one, 0)

        def drain_gathers(buf, sem):
            pltpu.make_async_copy(out_hbm.at[pl.ds(0, R)], buf, sem).wait()

        def fire_write(g, buf, sem):
            pltpu.async_copy(buf, out_hbm.at[pl.ds(row0 + g * R, R)], sem)

        def drain_write(buf, sem):
            pltpu.make_async_copy(out_hbm.at[pl.ds(0, R)], buf, sem).wait()

        fire_gathers(0, buf0, sg0)

        fire_gathers(1, buf1, sg1)
        drain_gathers(buf0, sg0)
        fire_write(0, buf0, sw0)
        drain_write(buf0, sw0)
        fire_gathers(2, buf0, sg0)
        drain_gathers(buf1, sg1)
        fire_write(1, buf1, sw1)

        def body(i, carry):
            g = 2 * i
            drain_write(buf1, sw1)
            fire_gathers(g + 1, buf1, sg1)
            drain_gathers(buf0, sg0)
            fire_write(g, buf0, sw0)
            drain_write(buf0, sw0)
            fire_gathers(g + 2, buf0, sg0)
            drain_gathers(buf1, sg1)
            fire_write(g + 1, buf1, sw1)
            return carry

        lax.fori_loop(1, G // 2 - 1, body, 0)

        drain_write(buf1, sw1)
        fire_gathers(G - 1, buf1, sg1)
        drain_gathers(buf0, sg0)
        fire_write(G - 2, buf0, sw0)
        drain_gathers(buf1, sg1)
        fire_write(G - 1, buf1, sw1)
        drain_write(buf0, sw0)
        drain_write(buf1, sw1)

    return k(idx, weight)


def kernel(idx, weight):
    N, Q = idx.shape
    D = weight.shape[1]
    return _sc_gather(idx.astype(jnp.int32), weight, N, Q, D)
